# baseline (speedup 1.0000x reference)
import jax
import jax.numpy as jnp
from jax.experimental import pallas as pl
from jax.experimental.pallas import tpu as pltpu

_EPS = 1e-12
_NCH = 16
_W = 1280
_HI = jax.lax.Precision.HIGHEST


def _box4(a):
    h = a[:, 0:1277] + a[:, 1:1278] + a[:, 2:1279] + a[:, 3:1280]
    return h[:, 0:1080] + h[:, 40:1120] + h[:, 80:1160] + h[:, 120:1200]


def _compact(a):
    r = a.shape[0]
    return a.reshape(r, 27, 40)[:, :, :37].reshape(r, 999)


def _body(x_ref, w_ref, cf_ref, c_ref, g_ref, l_ref,
          xn_s, sa_s, sv_s, y_s, tsq_s):
    t = pl.program_id(1)

    @pl.when(t == 0)
    def _setup():
        xm = x_ref[0]
        nrm = jnp.sqrt(jnp.sum(xm * xm, axis=0, keepdims=True))
        xn = xm / jnp.maximum(nrm, _EPS)
        xn_s[...] = xn
        logits = jax.lax.dot_general(
            w_ref[...], xn, (((1,), (0,)), ((), ())),
            preferred_element_type=jnp.float32, precision=_HI)
        mx = jnp.max(logits, axis=0, keepdims=True)
        e = jnp.exp(logits - mx)
        mask = (jax.lax.broadcasted_iota(jnp.int32, (1, _W), 1) < 1200)
        sa = jnp.where(mask, e / jnp.sum(e, axis=0, keepdims=True), 0.0)
        sa_s[...] = sa.reshape(_NCH, 4, _W)
        sv_s[...] = _box4(sa).reshape(_NCH, 4, 1080)
        m = jax.lax.dot_general(
            sa, xn, (((1,), (1,)), ((), ())),
            preferred_element_type=jnp.float32, precision=_HI)
        ssum = jnp.sum(sa, axis=1, keepdims=True)
        g = m - cf_ref[...] * ssum
        gn = g / jnp.maximum(
            jnp.sqrt(jnp.sum(g * g, axis=1, keepdims=True)), _EPS)
        tot = jnp.sqrt(jnp.sum(gn * gn))
        g_ref[...] = (gn / jnp.maximum(tot, _EPS)).reshape(1, 1, 8192)

    @pl.when(t < _NCH)
    def _compute():
        sa4 = sa_s[t]
        xn = xn_s[...]
        z = (sa4[:, None, :] * xn[None, :, :]).reshape(512, _W)
        v = _box4(z).reshape(4, 128, 1080)
        c4 = c_ref[0]
        s4 = sv_s[t]
        y = v - c4[:, :, None] * s4[:, None, :]
        nn = jnp.sqrt(jnp.sum(y * y, axis=1, keepdims=True))
        rcp = jnp.float32(0.0625) / jnp.maximum(
            nn * jnp.float32(0.0625), _EPS)
        y = (y * rcp).reshape(512, 1080)
        y_s[t] = y
        contrib = jnp.sum(y * y, axis=0, keepdims=True)
        prev = jnp.where(t == 0, jnp.zeros_like(contrib), tsq_s[...])
        tsq_s[...] = prev + contrib

    @pl.when(t >= _NCH)
    def _scale():
        j = t - _NCH
        inv = 1.0 / jnp.maximum(jnp.sqrt(_compact(tsq_s[...])), _EPS)
        out = _compact(y_s[j]) * inv
        l_ref[...] = out.reshape(1, 512, 999)


def kernel(x, conv_w, centroids):
    xr = jnp.pad(x.reshape(4, 128, 1200), ((0, 0), (0, 0), (0, _W - 1200)))
    cr = centroids.reshape(_NCH, 4, 128)
    vg, vl = pl.pallas_call(
        _body,
        grid=(4, 2 * _NCH),
        in_specs=[
            pl.BlockSpec((1, 128, _W), lambda n, t: (n, 0, 0)),
            pl.BlockSpec((64, 128), lambda n, t: (0, 0)),
            pl.BlockSpec((64, 128), lambda n, t: (0, 0)),
            pl.BlockSpec((1, 4, 128),
                         lambda n, t: (jnp.minimum(t, _NCH - 1), 0, 0)),
        ],
        out_specs=[
            pl.BlockSpec((1, 1, 8192), lambda n, t: (n, 0, 0)),
            pl.BlockSpec((1, 512, 999),
                         lambda n, t: (n, jnp.maximum(t - _NCH, 0), 0)),
        ],
        out_shape=[
            jax.ShapeDtypeStruct((4, 1, 8192), jnp.float32),
            jax.ShapeDtypeStruct((4, 8192, 999), jnp.float32),
        ],
        scratch_shapes=[
            pltpu.VMEM((128, _W), jnp.float32),
            pltpu.VMEM((_NCH, 4, _W), jnp.float32),
            pltpu.VMEM((_NCH, 4, 1080), jnp.float32),
            pltpu.VMEM((_NCH, 512, 1080), jnp.float32),
            pltpu.VMEM((1, 1080), jnp.float32),
        ],
        compiler_params=pltpu.CompilerParams(
            dimension_semantics=("arbitrary", "arbitrary"),
        ),
    )(xr, conv_w, centroids, cr)
    return vg.reshape(4, 8192), vl


# speedup vs baseline: 1.1358x; 1.1358x over previous
import jax
import jax.numpy as jnp
from jax.experimental import pallas as pl
from jax.experimental.pallas import tpu as pltpu

_EPS = 1e-12
_NCH = 16
_W = 1280
_HI = jax.lax.Precision.HIGHEST


def _box4(a):
    p = a[:, 0:1278] + a[:, 1:1279]
    h = p[:, 0:1276] + p[:, 2:1278]
    q = h[:, 0:1160] + h[:, 40:1200]
    return q[:, 0:1080] + q[:, 80:1160]


def _compact(a):
    r = a.shape[0]
    return a.reshape(r, 27, 40)[:, :, :37].reshape(r, 999)


def _body(x_ref, w_ref, cf_ref, c_ref, g_ref, l_ref,
          xn_s, sa_s, sv_s, y_s, tsq_s):
    t = pl.program_id(1)

    @pl.when(t == 0)
    def _setup():
        xm = x_ref[0]
        nrm = jnp.sqrt(jnp.sum(xm * xm, axis=0, keepdims=True))
        xn = xm / jnp.maximum(nrm, _EPS)
        xn_s[...] = xn
        logits = jax.lax.dot_general(
            w_ref[...], xn, (((1,), (0,)), ((), ())),
            preferred_element_type=jnp.float32, precision=_HI)
        mx = jnp.max(logits, axis=0, keepdims=True)
        e = jnp.exp(logits - mx)
        mask = (jax.lax.broadcasted_iota(jnp.int32, (1, _W), 1) < 1200)
        sa = jnp.where(mask, e / jnp.sum(e, axis=0, keepdims=True), 0.0)
        sa_s[...] = sa.reshape(_NCH, 4, _W)
        sv_s[...] = _box4(sa).reshape(_NCH, 4, 1080)
        m = jax.lax.dot_general(
            sa, xn, (((1,), (1,)), ((), ())),
            preferred_element_type=jnp.float32, precision=_HI)
        ssum = jnp.sum(sa, axis=1, keepdims=True)
        g = m - cf_ref[...] * ssum
        gn = g / jnp.maximum(
            jnp.sqrt(jnp.sum(g * g, axis=1, keepdims=True)), _EPS)
        tot = jnp.sqrt(jnp.sum(gn * gn))
        g_ref[...] = (gn / jnp.maximum(tot, _EPS)).reshape(1, 1, 8192)

    @pl.when(t < _NCH)
    def _compute():
        sa4 = sa_s[t]
        xn = xn_s[...]
        z = (sa4[:, None, :] * xn[None, :, :]).reshape(512, _W)
        v = _box4(z).reshape(4, 128, 1080)
        c4 = c_ref[0]
        s4 = sv_s[t]
        y = v - c4[:, :, None] * s4[:, None, :]
        nn = jnp.sqrt(jnp.sum(y * y, axis=1, keepdims=True))
        rcp = jnp.float32(0.0625) / jnp.maximum(
            nn * jnp.float32(0.0625), _EPS)
        y = (y * rcp).reshape(512, 1080)
        y_s[t] = y
        contrib = jnp.sum(y * y, axis=0, keepdims=True)
        prev = jnp.where(t == 0, jnp.zeros_like(contrib), tsq_s[...])
        tsq_s[...] = prev + contrib

    @pl.when(t >= _NCH)
    def _scale():
        j = t - _NCH
        inv = 1.0 / jnp.maximum(jnp.sqrt(_compact(tsq_s[...])), _EPS)
        out = _compact(y_s[j]) * inv
        l_ref[...] = out.reshape(1, 512, 999)


def kernel(x, conv_w, centroids):
    xr = jnp.pad(x.reshape(4, 128, 1200), ((0, 0), (0, 0), (0, _W - 1200)))
    cr = centroids.reshape(_NCH, 4, 128)
    vg, vl = pl.pallas_call(
        _body,
        grid=(4, 2 * _NCH),
        in_specs=[
            pl.BlockSpec((1, 128, _W), lambda n, t: (n, 0, 0)),
            pl.BlockSpec((64, 128), lambda n, t: (0, 0)),
            pl.BlockSpec((64, 128), lambda n, t: (0, 0)),
            pl.BlockSpec((1, 4, 128),
                         lambda n, t: (jnp.minimum(t, _NCH - 1), 0, 0)),
        ],
        out_specs=[
            pl.BlockSpec((1, 1, 8192), lambda n, t: (n, 0, 0)),
            pl.BlockSpec((1, 512, 999),
                         lambda n, t: (n, jnp.maximum(t - _NCH, 0), 0)),
        ],
        out_shape=[
            jax.ShapeDtypeStruct((4, 1, 8192), jnp.float32),
            jax.ShapeDtypeStruct((4, 8192, 999), jnp.float32),
        ],
        scratch_shapes=[
            pltpu.VMEM((128, _W), jnp.float32),
            pltpu.VMEM((_NCH, 4, _W), jnp.float32),
            pltpu.VMEM((_NCH, 4, 1080), jnp.float32),
            pltpu.VMEM((_NCH, 512, 1080), jnp.float32),
            pltpu.VMEM((1, 1080), jnp.float32),
        ],
        compiler_params=pltpu.CompilerParams(
            dimension_semantics=("arbitrary", "arbitrary"),
        ),
    )(xr, conv_w, centroids, cr)
    return vg.reshape(4, 8192), vl


# speedup vs baseline: 1.4393x; 1.2673x over previous
import jax
import jax.numpy as jnp
from jax.experimental import pallas as pl
from jax.experimental.pallas import tpu as pltpu

_EPS = 1e-12
_NCH = 16
_W = 1280
_HI = jax.lax.Precision.HIGHEST


def _box4(a):
    p = a[:, 0:1278] + a[:, 1:1279]
    h = p[:, 0:1276] + p[:, 2:1278]
    q = h[:, 0:1160] + h[:, 40:1200]
    return q[:, 0:1080] + q[:, 80:1160]


def _compact(a):
    return jnp.concatenate(
        [a[:, 40 * h:40 * h + 37] for h in range(27)], axis=1)


def _body(x_ref, w_ref, cf_ref, c_ref, g_ref, l_ref,
          xn_s, sa_s, sv_s, y_s, tsq_s):
    t = pl.program_id(1)

    @pl.when(t == 0)
    def _setup():
        xm = x_ref[0]
        nrm = jnp.sqrt(jnp.sum(xm * xm, axis=0, keepdims=True))
        xn = xm / jnp.maximum(nrm, _EPS)
        xn_s[...] = xn
        logits = jax.lax.dot_general(
            w_ref[...], xn, (((1,), (0,)), ((), ())),
            preferred_element_type=jnp.float32, precision=_HI)
        mx = jnp.max(logits, axis=0, keepdims=True)
        e = jnp.exp(logits - mx)
        mask = (jax.lax.broadcasted_iota(jnp.int32, (1, _W), 1) < 1200)
        sa = jnp.where(mask, e / jnp.sum(e, axis=0, keepdims=True), 0.0)
        sa_s[...] = sa.reshape(_NCH, 4, _W)
        sv_s[...] = _box4(sa).reshape(_NCH, 4, 1080)
        m = jax.lax.dot_general(
            sa, xn, (((1,), (1,)), ((), ())),
            preferred_element_type=jnp.float32, precision=_HI)
        ssum = jnp.sum(sa, axis=1, keepdims=True)
        g = m - cf_ref[...] * ssum
        gn = g / jnp.maximum(
            jnp.sqrt(jnp.sum(g * g, axis=1, keepdims=True)), _EPS)
        tot = jnp.sqrt(jnp.sum(gn * gn))
        g_ref[...] = (gn / jnp.maximum(tot, _EPS)).reshape(1, 1, 8192)

    @pl.when(t < _NCH)
    def _compute():
        sa4 = sa_s[t]
        xn = xn_s[...]
        z = (sa4[:, None, :] * xn[None, :, :]).reshape(512, _W)
        v = _box4(z).reshape(4, 128, 1080)
        c4 = c_ref[0]
        s4 = sv_s[t]
        y = v - c4[:, :, None] * s4[:, None, :]
        nn = jnp.sqrt(jnp.sum(y * y, axis=1, keepdims=True))
        rcp = jnp.float32(0.0625) / jnp.maximum(
            nn * jnp.float32(0.0625), _EPS)
        y = (y * rcp).reshape(512, 1080)
        y_s[t] = y
        contrib = jnp.sum(y * y, axis=0, keepdims=True)
        prev = jnp.where(t == 0, jnp.zeros_like(contrib), tsq_s[...])
        tsq_s[...] = prev + contrib

    @pl.when(t >= _NCH)
    def _scale():
        j = t - _NCH
        inv = 1.0 / jnp.maximum(jnp.sqrt(_compact(tsq_s[...])), _EPS)
        out = _compact(y_s[j]) * inv
        l_ref[...] = out.reshape(1, 512, 999)


def kernel(x, conv_w, centroids):
    xr = jnp.pad(x.reshape(4, 128, 1200), ((0, 0), (0, 0), (0, _W - 1200)))
    cr = centroids.reshape(_NCH, 4, 128)
    vg, vl = pl.pallas_call(
        _body,
        grid=(4, 2 * _NCH),
        in_specs=[
            pl.BlockSpec((1, 128, _W), lambda n, t: (n, 0, 0)),
            pl.BlockSpec((64, 128), lambda n, t: (0, 0)),
            pl.BlockSpec((64, 128), lambda n, t: (0, 0)),
            pl.BlockSpec((1, 4, 128),
                         lambda n, t: (jnp.minimum(t, _NCH - 1), 0, 0)),
        ],
        out_specs=[
            pl.BlockSpec((1, 1, 8192), lambda n, t: (n, 0, 0)),
            pl.BlockSpec((1, 512, 999),
                         lambda n, t: (n, jnp.maximum(t - _NCH, 0), 0)),
        ],
        out_shape=[
            jax.ShapeDtypeStruct((4, 1, 8192), jnp.float32),
            jax.ShapeDtypeStruct((4, 8192, 999), jnp.float32),
        ],
        scratch_shapes=[
            pltpu.VMEM((128, _W), jnp.float32),
            pltpu.VMEM((_NCH, 4, _W), jnp.float32),
            pltpu.VMEM((_NCH, 4, 1080), jnp.float32),
            pltpu.VMEM((_NCH, 512, 1080), jnp.float32),
            pltpu.VMEM((1, 1080), jnp.float32),
        ],
        compiler_params=pltpu.CompilerParams(
            dimension_semantics=("arbitrary", "arbitrary"),
        ),
    )(xr, conv_w, centroids, cr)
    return vg.reshape(4, 8192), vl


# speedup vs baseline: 1.4957x; 1.0391x over previous
import jax
import jax.numpy as jnp
from jax.experimental import pallas as pl
from jax.experimental.pallas import tpu as pltpu

_EPS = 1e-12
_NCH = 16
_W = 1280
_HI = jax.lax.Precision.HIGHEST


def _box4(a):
    p = a[:, 0:1278] + a[:, 1:1279]
    h = p[:, 0:1276] + p[:, 2:1278]
    q = h[:, 0:1160] + h[:, 40:1200]
    return q[:, 0:1080] + q[:, 80:1160]


def _compact(a):
    return jnp.concatenate(
        [a[:, 40 * h:40 * h + 37] for h in range(27)], axis=1)


def _body(x_ref, w_ref, cf_ref, c_ref, g_ref, l_ref,
          xn_s, sa_s, sv_s, y_s, rcp_s, tsq_s):
    t = pl.program_id(1)

    @pl.when(t == 0)
    def _setup():
        xm = x_ref[0]
        nrm = jnp.sqrt(jnp.sum(xm * xm, axis=0, keepdims=True))
        xn = xm / jnp.maximum(nrm, _EPS)
        xn_s[...] = xn
        logits = jax.lax.dot_general(
            w_ref[...], xn, (((1,), (0,)), ((), ())),
            preferred_element_type=jnp.float32, precision=_HI)
        mx = jnp.max(logits, axis=0, keepdims=True)
        e = jnp.exp(logits - mx)
        mask = (jax.lax.broadcasted_iota(jnp.int32, (1, _W), 1) < 1200)
        sa = jnp.where(mask, e / jnp.sum(e, axis=0, keepdims=True), 0.0)
        sa_s[...] = sa.reshape(_NCH, 4, _W)
        sv_s[...] = _box4(sa).reshape(_NCH, 4, 1080)
        m = jax.lax.dot_general(
            sa, xn, (((1,), (1,)), ((), ())),
            preferred_element_type=jnp.float32, precision=_HI)
        ssum = jnp.sum(sa, axis=1, keepdims=True)
        g = m - cf_ref[...] * ssum
        gn = g / jnp.maximum(
            jnp.sqrt(jnp.sum(g * g, axis=1, keepdims=True)), _EPS)
        tot = jnp.sqrt(jnp.sum(gn * gn))
        g_ref[...] = (gn / jnp.maximum(tot, _EPS)).reshape(1, 1, 8192)

    @pl.when(t < _NCH)
    def _compute():
        sa4 = sa_s[t]
        xn = xn_s[...]
        z = (sa4[:, None, :] * xn[None, :, :]).reshape(512, _W)
        p = z[:, 0:1278] + z[:, 1:1279]
        h = p[:, 0:1276] + p[:, 2:1278]
        q = h[:, 0:1160] + h[:, 40:1200]
        v = (q[:, 0:1080] + q[:, 80:1160]).reshape(4, 128, 1080)
        c4 = c_ref[0]
        s4 = sv_s[t]
        y = v - c4[:, :, None] * s4[:, None, :]
        y_s[t] = y.reshape(512, 1080)
        nsc = jnp.sqrt(jnp.sum(y * y, axis=1)) * jnp.float32(0.0625)
        rcp_s[t] = jnp.float32(0.0625) / jnp.maximum(nsc, _EPS)
        r = nsc / jnp.maximum(nsc, _EPS)
        contrib = jnp.sum(r * r, axis=0, keepdims=True)
        prev = jnp.where(t == 0, jnp.zeros_like(contrib), tsq_s[...])
        tsq_s[...] = prev + contrib

    @pl.when(t >= _NCH)
    def _scale():
        j = t - _NCH
        inv = 1.0 / jnp.maximum(jnp.sqrt(tsq_s[...]), _EPS)
        f = _compact(rcp_s[j] * inv)
        yc = _compact(y_s[j]).reshape(4, 128, 999)
        out = (yc * f[:, None, :]).reshape(512, 999)
        l_ref[...] = out.reshape(1, 512, 999)


def kernel(x, conv_w, centroids):
    xr = jnp.pad(x.reshape(4, 128, 1200), ((0, 0), (0, 0), (0, _W - 1200)))
    cr = centroids.reshape(_NCH, 4, 128)
    vg, vl = pl.pallas_call(
        _body,
        grid=(4, 2 * _NCH),
        in_specs=[
            pl.BlockSpec((1, 128, _W), lambda n, t: (n, 0, 0)),
            pl.BlockSpec((64, 128), lambda n, t: (0, 0)),
            pl.BlockSpec((64, 128), lambda n, t: (0, 0)),
            pl.BlockSpec((1, 4, 128),
                         lambda n, t: (jnp.minimum(t, _NCH - 1), 0, 0)),
        ],
        out_specs=[
            pl.BlockSpec((1, 1, 8192), lambda n, t: (n, 0, 0)),
            pl.BlockSpec((1, 512, 999),
                         lambda n, t: (n, jnp.maximum(t - _NCH, 0), 0)),
        ],
        out_shape=[
            jax.ShapeDtypeStruct((4, 1, 8192), jnp.float32),
            jax.ShapeDtypeStruct((4, 8192, 999), jnp.float32),
        ],
        scratch_shapes=[
            pltpu.VMEM((128, _W), jnp.float32),
            pltpu.VMEM((_NCH, 4, _W), jnp.float32),
            pltpu.VMEM((_NCH, 4, 1080), jnp.float32),
            pltpu.VMEM((_NCH, 512, 1080), jnp.float32),
            pltpu.VMEM((_NCH, 4, 1080), jnp.float32),
            pltpu.VMEM((1, 1080), jnp.float32),
        ],
        compiler_params=pltpu.CompilerParams(
            dimension_semantics=("arbitrary", "arbitrary"),
        ),
    )(xr, conv_w, centroids, cr)
    return vg.reshape(4, 8192), vl


# speedup vs baseline: 1.6304x; 1.0901x over previous
import jax
import jax.numpy as jnp
from jax.experimental import pallas as pl
from jax.experimental.pallas import tpu as pltpu

_EPS = 1e-12
_NCH = 8
_W = 1280
_HI = jax.lax.Precision.HIGHEST


def _box4(a):
    p = a[:, 0:1278] + a[:, 1:1279]
    h = p[:, 0:1276] + p[:, 2:1278]
    q = h[:, 0:1160] + h[:, 40:1200]
    return q[:, 0:1080] + q[:, 80:1160]


def _compact(a):
    return jnp.concatenate(
        [a[:, 40 * h:40 * h + 37] for h in range(27)], axis=1)


def _body(x_ref, w_ref, cf_ref, c_ref, g_ref, l_ref,
          xn_s, sa_s, y_s, rcp_s, tsq_s):
    t = pl.program_id(1)

    @pl.when(t == 0)
    def _setup():
        xm = x_ref[0]
        nrm = jnp.sqrt(jnp.sum(xm * xm, axis=0, keepdims=True))
        xn = xm / jnp.maximum(nrm, _EPS)
        xn_s[...] = xn
        logits = jax.lax.dot_general(
            w_ref[...], xn, (((1,), (0,)), ((), ())),
            preferred_element_type=jnp.float32, precision=_HI)
        mx = jnp.max(logits, axis=0, keepdims=True)
        e = jnp.exp(logits - mx)
        mask = (jax.lax.broadcasted_iota(jnp.int32, (1, _W), 1) < 1200)
        sa = jnp.where(mask, e / jnp.sum(e, axis=0, keepdims=True), 0.0)
        sa_s[...] = sa.reshape(_NCH, 8, _W)
        m = jax.lax.dot_general(
            sa, xn, (((1,), (1,)), ((), ())),
            preferred_element_type=jnp.float32, precision=_HI)
        ssum = jnp.sum(sa, axis=1, keepdims=True)
        g = m - cf_ref[...] * ssum
        gn = g / jnp.maximum(
            jnp.sqrt(jnp.sum(g * g, axis=1, keepdims=True)), _EPS)
        tot = jnp.sqrt(jnp.sum(gn * gn))
        g_ref[...] = (gn / jnp.maximum(tot, _EPS)).reshape(1, 1, 8192)

    @pl.when(t < _NCH)
    def _compute():
        sa8 = sa_s[t]
        xn = xn_s[...]
        c8 = c_ref[0]
        z = ((xn[None, :, :] - c8[:, :, None])
             * sa8[:, None, :]).reshape(1024, _W)
        p = z[:, 0:1278] + z[:, 1:1279]
        h = p[:, 0:1276] + p[:, 2:1278]
        q = h[:, 0:1160] + h[:, 40:1200]
        y = (q[:, 0:1080] + q[:, 80:1160]).reshape(8, 128, 1080)
        y_s[t] = y.reshape(1024, 1080).astype(jnp.bfloat16)
        nsc = jnp.sqrt(jnp.sum(y * y, axis=1)) * jnp.float32(0.0625)
        rcp_s[t] = jnp.float32(0.0625) / jnp.maximum(nsc, _EPS)
        r = nsc / jnp.maximum(nsc, _EPS)
        contrib = jnp.sum(r * r, axis=0, keepdims=True)
        prev = jnp.where(t == 0, jnp.zeros_like(contrib), tsq_s[...])
        tsq_s[...] = prev + contrib

    @pl.when(t >= _NCH)
    def _scale():
        j = t - _NCH
        inv = 1.0 / jnp.maximum(jnp.sqrt(tsq_s[...]), _EPS)
        f = rcp_s[j] * inv
        prod = (y_s[j].astype(jnp.float32).reshape(8, 128, 1080)
                * f[:, None, :]).reshape(1024, 1080)
        l_ref[...] = _compact(prod).reshape(1, 1024, 999)


def kernel(x, conv_w, centroids):
    xr = jnp.pad(x.reshape(4, 128, 1200), ((0, 0), (0, 0), (0, _W - 1200)))
    cr = centroids.reshape(_NCH, 8, 128)
    vg, vl = pl.pallas_call(
        _body,
        grid=(4, 2 * _NCH),
        in_specs=[
            pl.BlockSpec((1, 128, _W), lambda n, t: (n, 0, 0)),
            pl.BlockSpec((64, 128), lambda n, t: (0, 0)),
            pl.BlockSpec((64, 128), lambda n, t: (0, 0)),
            pl.BlockSpec((1, 8, 128),
                         lambda n, t: (jnp.minimum(t, _NCH - 1), 0, 0)),
        ],
        out_specs=[
            pl.BlockSpec((1, 1, 8192), lambda n, t: (n, 0, 0)),
            pl.BlockSpec((1, 1024, 999),
                         lambda n, t: (n, jnp.maximum(t - _NCH, 0), 0)),
        ],
        out_shape=[
            jax.ShapeDtypeStruct((4, 1, 8192), jnp.float32),
            jax.ShapeDtypeStruct((4, 8192, 999), jnp.float32),
        ],
        scratch_shapes=[
            pltpu.VMEM((128, _W), jnp.float32),
            pltpu.VMEM((_NCH, 8, _W), jnp.float32),
            pltpu.VMEM((_NCH, 1024, 1080), jnp.bfloat16),
            pltpu.VMEM((_NCH, 8, 1080), jnp.float32),
            pltpu.VMEM((1, 1080), jnp.float32),
        ],
        compiler_params=pltpu.CompilerParams(
            dimension_semantics=("arbitrary", "arbitrary"),
        ),
    )(xr, conv_w, centroids, cr)
    return vg.reshape(4, 8192), vl


# speedup vs baseline: 1.8753x; 1.1502x over previous
import jax
import jax.numpy as jnp
from jax.experimental import pallas as pl
from jax.experimental.pallas import tpu as pltpu

_EPS = 1e-12
_NCH = 8
_W = 1280
_HI = jax.lax.Precision.HIGHEST


def _box4(a):
    p = a[:, 0:1278] + a[:, 1:1279]
    h = p[:, 0:1276] + p[:, 2:1278]
    q = h[:, 0:1160] + h[:, 40:1200]
    return q[:, 0:1080] + q[:, 80:1160]


def _compact(a):
    return jnp.concatenate(
        [a[:, 40 * h:40 * h + 37] for h in range(27)], axis=1)


def _body(x_ref, w_ref, cf_ref, c_ref, g_ref, l_ref,
          xn_s, sa_s, y_s, rcp_s, tsq_s):
    t = pl.program_id(1)

    @pl.when(t == 0)
    def _setup():
        xm = x_ref[0]
        nrm = jnp.sqrt(jnp.sum(xm * xm, axis=0, keepdims=True))
        xn = xm / jnp.maximum(nrm, _EPS)
        xn_s[...] = xn
        logits = jax.lax.dot_general(
            w_ref[...], xn, (((1,), (0,)), ((), ())),
            preferred_element_type=jnp.float32, precision=_HI)
        mx = jnp.max(logits, axis=0, keepdims=True)
        e = jnp.exp(logits - mx)
        mask = (jax.lax.broadcasted_iota(jnp.int32, (1, _W), 1) < 1200)
        sa = jnp.where(mask, e / jnp.sum(e, axis=0, keepdims=True), 0.0)
        sa_s[...] = sa.reshape(_NCH, 8, _W)
        m = jax.lax.dot_general(
            sa, xn, (((1,), (1,)), ((), ())),
            preferred_element_type=jnp.float32, precision=_HI)
        ssum = jnp.sum(sa, axis=1, keepdims=True)
        g = m - cf_ref[...] * ssum
        gn = g / jnp.maximum(
            jnp.sqrt(jnp.sum(g * g, axis=1, keepdims=True)), _EPS)
        tot = jnp.sqrt(jnp.sum(gn * gn))
        g_ref[...] = (gn / jnp.maximum(tot, _EPS)).reshape(1, 1, 8192)

    @pl.when(t < _NCH)
    def _compute():
        sa8 = sa_s[t]
        xn = xn_s[...]
        c8 = c_ref[0]
        z = ((xn[None, :, :] - c8[:, :, None])
             * sa8[:, None, :]).reshape(1024, _W)
        p = z[:, 0:1278] + z[:, 1:1279]
        h = p[:, 0:1276] + p[:, 2:1278]
        q = h[:, 0:1160] + h[:, 40:1200]
        y = (q[:, 0:1080] + q[:, 80:1160]).reshape(8, 128, 1080)
        y_s[t] = y.reshape(1024, 1080).astype(jnp.bfloat16)
        nsc = jnp.sqrt(jnp.sum(y * y, axis=1)) * jnp.float32(0.0625)
        rcp_s[t] = jnp.float32(0.0625) / jnp.maximum(nsc, _EPS)
        r = nsc / jnp.maximum(nsc, _EPS)
        contrib = jnp.sum(r * r, axis=0, keepdims=True)
        prev = jnp.where(t == 0, jnp.zeros_like(contrib), tsq_s[...])
        tsq_s[...] = prev + contrib

    @pl.when(t >= _NCH)
    def _scale():
        j = t - _NCH
        inv = 1.0 / jnp.maximum(jnp.sqrt(tsq_s[...]), _EPS)
        f = _compact(rcp_s[j] * inv)
        yb = _compact(y_s[j]).astype(jnp.float32)
        prod = (yb.reshape(8, 128, 999) * f[:, None, :])
        l_ref[...] = prod.reshape(1, 1024, 999)


def kernel(x, conv_w, centroids):
    xr = jnp.pad(x.reshape(4, 128, 1200), ((0, 0), (0, 0), (0, _W - 1200)))
    cr = centroids.reshape(_NCH, 8, 128)
    vg, vl = pl.pallas_call(
        _body,
        grid=(4, 2 * _NCH),
        in_specs=[
            pl.BlockSpec((1, 128, _W), lambda n, t: (n, 0, 0)),
            pl.BlockSpec((64, 128), lambda n, t: (0, 0)),
            pl.BlockSpec((64, 128), lambda n, t: (0, 0)),
            pl.BlockSpec((1, 8, 128),
                         lambda n, t: (jnp.minimum(t, _NCH - 1), 0, 0)),
        ],
        out_specs=[
            pl.BlockSpec((1, 1, 8192), lambda n, t: (n, 0, 0)),
            pl.BlockSpec((1, 1024, 999),
                         lambda n, t: (n, jnp.maximum(t - _NCH, 0), 0)),
        ],
        out_shape=[
            jax.ShapeDtypeStruct((4, 1, 8192), jnp.float32),
            jax.ShapeDtypeStruct((4, 8192, 999), jnp.float32),
        ],
        scratch_shapes=[
            pltpu.VMEM((128, _W), jnp.float32),
            pltpu.VMEM((_NCH, 8, _W), jnp.float32),
            pltpu.VMEM((_NCH, 1024, 1080), jnp.bfloat16),
            pltpu.VMEM((_NCH, 8, 1080), jnp.float32),
            pltpu.VMEM((1, 1080), jnp.float32),
        ],
        compiler_params=pltpu.CompilerParams(
            dimension_semantics=("arbitrary", "arbitrary"),
        ),
    )(xr, conv_w, centroids, cr)
    return vg.reshape(4, 8192), vl
